# baseline (speedup 1.0000x reference)
import jax
import jax.numpy as jnp
from jax import lax
from jax.experimental import pallas as pl
from jax.experimental.pallas import tpu as pltpu
from jax.experimental.pallas import tpu_sc as plsc

N_ROWS = 160000
N_FEATS = 256
N_SEG = 10000

NUM_CORES = 2
NUM_SUBCORES = 16
LANES = 16
NV = 8

DHALF = N_FEATS // NUM_CORES
ROWS_PER_TILE = N_ROWS // NUM_SUBCORES
SUB = 80
N_CHUNK = ROWS_PER_TILE // SUB
IDX_PAD = 128
NRING = 2
OB = 32

SEG_PER_TILE = 640
S_PAD = SEG_PER_TILE * NUM_SUBCORES
TRASH = N_SEG + 100
SEG_CHUNK = 40
FULL_SEG_CHUNKS = SEG_PER_TILE // SEG_CHUNK
LAST_SEG_CHUNKS = (N_SEG - 15 * SEG_PER_TILE) // SEG_CHUNK

CNT_BATCH = 25


def _counts_body(batch_hbm, cnt_hbm, ibuf, ones, zbuf, cnt_sh, csem):
  core = lax.axis_index("c")
  tile = lax.axis_index("s")
  seg_base = tile * SEG_PER_TILE

  zeros16 = jnp.zeros((LANES,), jnp.float32)
  ones16 = jnp.ones((LANES,), jnp.float32)

  def init_rows(s, _):
    zbuf[s, :] = zeros16
    ones[s, :] = ones16
    return _
  lax.fori_loop(0, SUB, init_rows, 0)

  for k in range(SEG_PER_TILE // SUB):
    pltpu.sync_copy(zbuf, cnt_sh.at[pl.ds(seg_base + k * SUB, SUB)])

  plsc.subcore_barrier()

  pltpu.sync_copy(batch_hbm.at[tile], ibuf)
  groups = {}
  for g in range(N_CHUNK // CNT_BATCH):
    if g >= 1:
      for d in groups.pop(g - 1):
        d.wait()
    descs = []
    for j in range(CNT_BATCH):
      descs.append(pltpu.async_copy(
          ones, cnt_sh.at[ibuf.at[g * CNT_BATCH + j]], csem, add=True))
    groups[g] = descs
  for g in sorted(groups):
    for d in groups.pop(g):
      d.wait()

  plsc.subcore_barrier()

  @pl.when(core == 0)
  def _():
    pltpu.sync_copy(cnt_sh.at[pl.ds(seg_base, SEG_PER_TILE)],
                    cnt_hbm.at[pl.ds(seg_base, SEG_PER_TILE)])


def _pool_body(feats_hbm, batch_hbm, cnt_hbm, out_hbm, fbuf, ibuf, sbuf,
               cbuf, obuf, oseg, sums_sh, lsem, ssem):
  core = lax.axis_index("c")
  tile = lax.axis_index("s")
  col0 = core * DHALF
  row_base = tile * ROWS_PER_TILE
  seg_base = tile * SEG_PER_TILE

  zeros16 = jnp.zeros((LANES,), jnp.float32)
  ones16 = jnp.ones((LANES,), jnp.float32)
  lane0 = lax.iota(jnp.int32, LANES) == 0

  def zero_sbuf(s, _):
    for v in range(NV):
      sbuf[s, pl.ds(v * LANES, LANES)] = zeros16
    return _
  lax.fori_loop(0, SEG_CHUNK, zero_sbuf, 0)

  for k in range(FULL_SEG_CHUNKS):
    base = pl.multiple_of(seg_base + k * SEG_CHUNK, 8)
    pltpu.sync_copy(sbuf, sums_sh.at[pl.ds(base, SEG_CHUNK)])

  plsc.subcore_barrier()

  def feats_src(k):
    row0 = pl.multiple_of(row_base + k * SUB, 8)
    return feats_hbm.at[pl.ds(row0, SUB), pl.ds(col0, DHALF)]

  def start_load(k):
    slot = lax.rem(k, NRING)
    return pltpu.async_copy(feats_src(k), fbuf.at[pl.ds(slot * SUB, SUB)],
                            lsem)

  def store_oseg(s2, e, seg):
    plsc.store_scatter(oseg, [jnp.full((LANES,), s2, jnp.int32),
                              jnp.full((LANES,), e, jnp.int32)],
                       jnp.full((LANES,), seg, jnp.int32), mask=lane0)

  def drain_obuf(s2):
    pltpu.make_async_copy(obuf.at[pl.ds(s2 * OB, OB)],
                          sums_sh.at[oseg.at[s2]], ssem).wait()

  def fire_obuf(s2):
    pltpu.async_copy(obuf.at[pl.ds(s2 * OB, OB)],
                     sums_sh.at[oseg.at[s2]], ssem, add=True)

  def emit(prev, nout, acc):
    e = lax.rem(nout, OB)
    s2 = lax.rem(lax.div(nout, OB), 2)
    fills = lax.div(nout, OB)
    eseg = jnp.where(prev < 0, TRASH, prev)

    @pl.when(jnp.logical_and(e == 0, fills >= 2))
    def _():
      drain_obuf(s2)

    orow = jnp.full((LANES,), s2 * OB + e, jnp.int32)
    cols = lax.iota(jnp.int32, LANES)
    for v in range(NV):
      plsc.store_scatter(obuf, [orow, cols + v * LANES], acc[v])
    store_oseg(s2, e, eseg)

    @pl.when(lax.rem(nout + 1, OB) == 0)
    def _():
      fire_obuf(s2)

  start_load(0)

  def chunk_body(k, state):
    slot = lax.rem(k, NRING)
    pltpu.make_async_copy(feats_src(k), fbuf.at[pl.ds(slot * SUB, SUB)],
                          lsem).wait()

    @pl.when(k + 1 < N_CHUNK)
    def _prefetch():
      start_load(k + 1)

    @pl.when(lax.rem(k, 8) == 0)
    def _idx():
      pltpu.sync_copy(batch_hbm.at[tile].at[pl.ds(pl.multiple_of(k, 8), 8)],
                      ibuf)


    def group_body(g, st):
      iv = ibuf[lax.rem(k, 8) * SUB // SUB, pl.ds(g * LANES, LANES)] if False else ibuf[lax.rem(k, 8), pl.ds(g * LANES, LANES)]
      prev, nout = st[0], st[1]
      acc = list(st[2:])
      for l in range(LANES):
        seg = iv[l]
        same = seg == prev
        ridx = g * LANES + l
        row = [fbuf[slot * SUB + ridx, pl.ds(v * LANES, LANES)]
               for v in range(NV)]

        @pl.when(jnp.logical_not(same))
        def _(prev=prev, nout=nout, acc=tuple(acc)):
          emit(prev, nout, acc)

        acc = [jnp.where(same, acc[v] + row[v], row[v]) for v in range(NV)]
        nout = jnp.where(same, nout, nout + 1)
        prev = seg
      return (prev, nout) + tuple(acc)

    return lax.fori_loop(0, SUB // LANES, group_body, state)

  init = (jnp.int32(-1), jnp.int32(0)) + tuple(zeros16 for _ in range(NV))
  final = lax.fori_loop(0, N_CHUNK, chunk_body, init)
  prev, nout = final[0], final[1]
  acc = list(final[2:])

  emit(prev, nout, acc)
  nout = nout + 1
  e2 = lax.rem(nout, OB)
  s2c = lax.rem(lax.div(nout, OB), 2)

  @pl.when(e2 != 0)
  def _pad_and_fire():
    for ee in range(1, OB):
      @pl.when(ee >= e2)
      def _():
        store_oseg(s2c, ee, TRASH)
    fire_obuf(s2c)

  fills = lax.div(nout, OB) + jnp.where(e2 != 0, 1, 0)

  @pl.when(fills >= 1)
  def _():
    drain_obuf(lax.rem(fills - 1, 2))

  @pl.when(fills >= 2)
  def _():
    drain_obuf(lax.rem(fills - 2, 2))

  plsc.subcore_barrier()

  n_chunks = jnp.where(tile == NUM_SUBCORES - 1, LAST_SEG_CHUNKS,
                       FULL_SEG_CHUNKS)

  def finalize(k, _):
    base = pl.multiple_of(seg_base + k * SEG_CHUNK, 8)
    pltpu.sync_copy(sums_sh.at[pl.ds(base, SEG_CHUNK)], sbuf)
    pltpu.sync_copy(cnt_hbm.at[pl.ds(base, SEG_CHUNK)], cbuf)

    def div_one(s, _2):
      cntv = cbuf[s, :]
      inv = (ones16 / jnp.maximum(cntv, ones16))[0]
      for v in range(NV):
        sl = pl.ds(v * LANES, LANES)
        sbuf[s, sl] = sbuf[s, sl] * inv
      return _2
    lax.fori_loop(0, SEG_CHUNK, div_one, 0)

    pltpu.sync_copy(sbuf, out_hbm.at[pl.ds(base, SEG_CHUNK),
                                     pl.ds(col0, DHALF)])
    return _
  lax.fori_loop(0, n_chunks, finalize, 0)


@jax.jit
def _pool_mean(feats, batch3d):
  mesh = plsc.VectorSubcoreMesh(core_axis_name="c", subcore_axis_name="s")
  counts = pl.kernel(
      _counts_body,
      out_type=jax.ShapeDtypeStruct((S_PAD, LANES), jnp.float32),
      mesh=mesh,
      compiler_params=pltpu.CompilerParams(use_tc_tiling_on_sc=False),
      scratch_types=[
          pltpu.VMEM((IDX_PAD, SUB), jnp.int32),
          pltpu.VMEM((SUB, LANES), jnp.float32),
          pltpu.VMEM((SUB, LANES), jnp.float32),
          pltpu.VMEM_SHARED((S_PAD, LANES), jnp.float32),
          pltpu.SemaphoreType.DMA,
      ],
  )(batch3d)
  return pl.kernel(
      _pool_body,
      out_type=jax.ShapeDtypeStruct((N_SEG, N_FEATS), jnp.float32),
      mesh=mesh,
      compiler_params=pltpu.CompilerParams(needs_layout_passes=False),
      scratch_types=[
          pltpu.VMEM((NRING * SUB, DHALF), jnp.float32),
          pltpu.VMEM((8, SUB), jnp.int32),
          pltpu.VMEM((SEG_CHUNK, DHALF), jnp.float32),
          pltpu.VMEM((SEG_CHUNK, LANES), jnp.float32),
          pltpu.VMEM((2 * OB, DHALF), jnp.float32),
          pltpu.VMEM((2, OB), jnp.int32),
          pltpu.VMEM_SHARED((S_PAD, DHALF), jnp.float32),
          pltpu.SemaphoreType.DMA,
          pltpu.SemaphoreType.DMA,
      ],
  )(feats, batch3d, counts)


def kernel(feats, batch):
  batch3d = batch.astype(jnp.int32).reshape(NUM_SUBCORES, N_CHUNK, SUB)
  batch3d = jnp.pad(batch3d, ((0, 0), (0, IDX_PAD - N_CHUNK), (0, 0)),
                    constant_values=TRASH)
  return _pool_mean(feats, batch3d)


# speedup vs baseline: 1.4265x; 1.4265x over previous
import jax
import jax.numpy as jnp
from jax import lax
from jax.experimental import pallas as pl
from jax.experimental.pallas import tpu as pltpu
from jax.experimental.pallas import tpu_sc as plsc

N_ROWS = 160000
N_FEATS = 256
N_SEG = 10000

NUM_CORES = 2
NUM_SUBCORES = 16
LANES = 16

DHALF = N_FEATS // NUM_CORES
ROWS_PER_TILE = N_ROWS // NUM_SUBCORES
SUB = 80
N_CHUNK = ROWS_PER_TILE // SUB
NRING = 3
IDX_PAD = 128

SEG_PER_TILE = 640
S_PAD = SEG_PER_TILE * NUM_SUBCORES
SEG_CHUNK = 40
FULL_SEG_CHUNKS = SEG_PER_TILE // SEG_CHUNK
LAST_SEG_CHUNKS = (N_SEG - 15 * SEG_PER_TILE) // SEG_CHUNK

CNT_BATCH = 25


def _counts_body(batch_hbm, cnt_hbm, ibuf, ones, zbuf, cnt_sh, csem):
  core = lax.axis_index("c")
  tile = lax.axis_index("s")
  seg_base = tile * SEG_PER_TILE

  zeros16 = jnp.zeros((LANES,), jnp.float32)
  ones16 = jnp.ones((LANES,), jnp.float32)

  def init_rows(s, _):
    zbuf[s, :] = zeros16
    ones[s, :] = ones16
    return _
  lax.fori_loop(0, SUB, init_rows, 0)

  for k in range(SEG_PER_TILE // SUB):
    pltpu.sync_copy(zbuf, cnt_sh.at[pl.ds(seg_base + k * SUB, SUB)])

  plsc.subcore_barrier()

  pltpu.sync_copy(batch_hbm.at[tile], ibuf)
  groups = {}
  for g in range(N_CHUNK // CNT_BATCH):
    if g >= 1:
      for d in groups.pop(g - 1):
        d.wait()
    descs = []
    for j in range(CNT_BATCH):
      descs.append(pltpu.async_copy(
          ones, cnt_sh.at[ibuf.at[g * CNT_BATCH + j]], csem, add=True))
    groups[g] = descs
  for g in sorted(groups):
    for d in groups.pop(g):
      d.wait()

  plsc.subcore_barrier()

  @pl.when(core == 0)
  def _():
    pltpu.sync_copy(cnt_sh.at[pl.ds(seg_base, SEG_PER_TILE)],
                    cnt_hbm.at[pl.ds(seg_base, SEG_PER_TILE)])


def _pool_body(feats_hbm, batch_hbm, cnt_hbm, out_hbm, fbuf, ibuf, sbuf,
               cbuf, sums_sh, lsem, ssem):
  core = lax.axis_index("c")
  tile = lax.axis_index("s")
  col0 = core * DHALF
  row_base = tile * ROWS_PER_TILE
  seg_base = tile * SEG_PER_TILE

  zeros16 = jnp.zeros((LANES,), jnp.float32)
  ones16 = jnp.ones((LANES,), jnp.float32)

  def zero_sbuf(s, _):
    for v in range(DHALF // LANES):
      sbuf[s, pl.ds(v * LANES, LANES)] = zeros16
    return _
  lax.fori_loop(0, SEG_CHUNK, zero_sbuf, 0)

  for k in range(FULL_SEG_CHUNKS):
    base = pl.multiple_of(seg_base + k * SEG_CHUNK, 8)
    pltpu.sync_copy(sbuf, sums_sh.at[pl.ds(base, SEG_CHUNK)])

  plsc.subcore_barrier()

  def feats_src(k):
    row0 = pl.multiple_of(row_base + k * SUB, 8)
    return feats_hbm.at[pl.ds(row0, SUB), pl.ds(col0, DHALF)]

  def start_load(k):
    slot = lax.rem(k, NRING)
    return pltpu.async_copy(feats_src(k), fbuf.at[slot], lsem)

  start_load(0)
  start_load(1)

  def accum(k, _):
    slot = lax.rem(k, NRING)

    @pl.when(lax.rem(k, 8) == 0)
    def _idx():
      pltpu.sync_copy(batch_hbm.at[tile].at[pl.ds(pl.multiple_of(k, 8), 8)],
                      ibuf)

    pltpu.make_async_copy(feats_src(k), fbuf.at[slot], lsem).wait()

    @pl.when(k >= 2)
    def _drain():
      prev = lax.rem(k - 2, NRING)
      pltpu.make_async_copy(fbuf.at[prev],
                            sums_sh.at[ibuf.at[lax.rem(k - 2, 8)]],
                            ssem).wait()

    @pl.when(k + 2 < N_CHUNK)
    def _prefetch():
      start_load(k + 2)

    pltpu.async_copy(fbuf.at[slot], sums_sh.at[ibuf.at[lax.rem(k, 8)]],
                     ssem, add=True)
    return _
  lax.fori_loop(0, N_CHUNK, accum, 0)
  for kk in (N_CHUNK - 2, N_CHUNK - 1):
    pltpu.make_async_copy(fbuf.at[lax.rem(kk, NRING)],
                          sums_sh.at[ibuf.at[lax.rem(kk, 8)]], ssem).wait()

  plsc.subcore_barrier()

  n_chunks = jnp.where(tile == NUM_SUBCORES - 1, LAST_SEG_CHUNKS,
                       FULL_SEG_CHUNKS)

  def finalize(k, _):
    base = pl.multiple_of(seg_base + k * SEG_CHUNK, 8)
    pltpu.sync_copy(sums_sh.at[pl.ds(base, SEG_CHUNK)], sbuf)
    pltpu.sync_copy(cnt_hbm.at[pl.ds(base, SEG_CHUNK)], cbuf)

    def div_one(s, _2):
      cntv = cbuf[s, :]
      inv = (ones16 / jnp.maximum(cntv, ones16))[0]
      for v in range(DHALF // LANES):
        sl = pl.ds(v * LANES, LANES)
        sbuf[s, sl] = sbuf[s, sl] * inv
      return _2
    lax.fori_loop(0, SEG_CHUNK, div_one, 0)

    pltpu.sync_copy(sbuf, out_hbm.at[pl.ds(base, SEG_CHUNK),
                                     pl.ds(col0, DHALF)])
    return _
  lax.fori_loop(0, n_chunks, finalize, 0)


@jax.jit
def _pool_mean(feats, batch3d):
  mesh = plsc.VectorSubcoreMesh(core_axis_name="c", subcore_axis_name="s")
  counts = pl.kernel(
      _counts_body,
      out_type=jax.ShapeDtypeStruct((S_PAD, LANES), jnp.float32),
      mesh=mesh,
      compiler_params=pltpu.CompilerParams(use_tc_tiling_on_sc=False),
      scratch_types=[
          pltpu.VMEM((IDX_PAD, SUB), jnp.int32),
          pltpu.VMEM((SUB, LANES), jnp.float32),
          pltpu.VMEM((SUB, LANES), jnp.float32),
          pltpu.VMEM_SHARED((S_PAD, LANES), jnp.float32),
          pltpu.SemaphoreType.DMA,
      ],
  )(batch3d)
  return pl.kernel(
      _pool_body,
      out_type=jax.ShapeDtypeStruct((N_SEG, N_FEATS), jnp.float32),
      mesh=mesh,
      scratch_types=[
          pltpu.VMEM((NRING, SUB, DHALF), jnp.float32),
          pltpu.VMEM((8, SUB), jnp.int32),
          pltpu.VMEM((SEG_CHUNK, DHALF), jnp.float32),
          pltpu.VMEM((SEG_CHUNK, LANES), jnp.float32),
          pltpu.VMEM_SHARED((S_PAD, DHALF), jnp.float32),
          pltpu.SemaphoreType.DMA,
          pltpu.SemaphoreType.DMA,
      ],
  )(feats, batch3d, counts)


def kernel(feats, batch):
  batch3d = batch.astype(jnp.int32).reshape(NUM_SUBCORES, N_CHUNK, SUB)
  batch3d = jnp.pad(batch3d, ((0, 0), (0, IDX_PAD - N_CHUNK), (0, 0)))
  return _pool_mean(feats, batch3d)
